# Initial kernel scaffold; baseline (speedup 1.0000x reference)
import jax
import jax.numpy as jnp
from jax.experimental import pallas as pl


def kernel(x, edge_index, W1_l, W1_r, b1, W2_l, W2_r, b2):
    raise NotImplementedError("write your pallas kernel here")


# speedup vs baseline: 7.0173x; 7.0173x over previous
import functools

import jax
import jax.numpy as jnp
from jax import lax
from jax.experimental import pallas as pl
from jax.experimental.pallas import tpu as pltpu
from jax.experimental.pallas import tpu_sc as plsc

N_NODES = 10000
N_EDGES = 320000
D = 128

NC = 2
NS = 16
NW = NC * NS

CH = 80
G = 5
E_PER_W = N_EDGES // NW
NG = E_PER_W // (G * CH)
ROWS_PER_TILE = N_NODES // NS
ZCH = 125
CBLK = 1000
NPAD = 10240

_MESH = plsc.VectorSubcoreMesh(core_axis_name="c", subcore_axis_name="s")


def _sc_agg_cnt_body(x_hbm, srcs_hbm, dsts_hbm, zeros_hbm, out_agg, out_cnt,
                     src_g, dst_g, rows_v, cnt_v, agg_sh, sem):
    c = lax.axis_index("c")
    s = lax.axis_index("s")
    wid = c * NS + s
    for k in range(ROWS_PER_TILE // ZCH):
        sl = pl.ds(s * ROWS_PER_TILE + k * ZCH, ZCH)
        pltpu.sync_copy(zeros_hbm, agg_sh.at[sl])
    zeros16 = jnp.zeros((16,), jnp.float32)

    def zstep(v, carry):
        cnt_v[pl.ds(v * 16, 16)] = zeros16
        return carry

    lax.fori_loop(0, NPAD // 16, zstep, 0)
    plsc.subcore_barrier()
    ones16 = jnp.ones((16,), jnp.float32)

    def step(g, carry):
        pltpu.sync_copy(srcs_hbm.at[wid, g], src_g)
        pltpu.sync_copy(dsts_hbm.at[wid, g], dst_g)
        for k in range(G):
            pltpu.async_copy(x_hbm.at[src_g.at[k]], rows_v, sem).wait()
            pltpu.sync_copy(rows_v, agg_sh.at[dst_g.at[k]], add=True)
            for v in range(CH // 16):
                idx = dst_g[k, pl.ds(v * 16, 16)]
                plsc.addupdate_scatter(cnt_v, [idx], ones16)
        return carry

    lax.fori_loop(0, NG, step, 0)
    plsc.subcore_barrier()
    pltpu.sync_copy(cnt_v, out_cnt.at[wid])
    @pl.when(s == 0)
    def _():
        pltpu.sync_copy(agg_sh, out_agg.at[c])


def _sc_agg_body(x_hbm, srcs_hbm, dsts_hbm, zeros_hbm, out_agg,
                 src_g, dst_g, rows_v, agg_sh, sem):
    c = lax.axis_index("c")
    s = lax.axis_index("s")
    wid = c * NS + s
    for k in range(ROWS_PER_TILE // ZCH):
        sl = pl.ds(s * ROWS_PER_TILE + k * ZCH, ZCH)
        pltpu.sync_copy(zeros_hbm, agg_sh.at[sl])
    plsc.subcore_barrier()

    def step(g, carry):
        pltpu.sync_copy(srcs_hbm.at[wid, g], src_g)
        pltpu.sync_copy(dsts_hbm.at[wid, g], dst_g)
        for k in range(G):
            pltpu.async_copy(x_hbm.at[src_g.at[k]], rows_v, sem).wait()
            pltpu.sync_copy(rows_v, agg_sh.at[dst_g.at[k]], add=True)
        return carry

    lax.fori_loop(0, NG, step, 0)
    plsc.subcore_barrier()
    @pl.when(s == 0)
    def _():
        pltpu.sync_copy(agg_sh, out_agg.at[c])


_sc_agg_cnt = pl.kernel(
    _sc_agg_cnt_body,
    out_type=(jax.ShapeDtypeStruct((NC, N_NODES, D), jnp.float32),
              jax.ShapeDtypeStruct((NW, NPAD), jnp.float32)),
    mesh=_MESH,
    compiler_params=pltpu.CompilerParams(needs_layout_passes=False),
    scratch_types=[
        pltpu.VMEM((G, CH), jnp.int32),
        pltpu.VMEM((G, CH), jnp.int32),
        pltpu.VMEM((CH, D), jnp.float32),
        pltpu.VMEM((NPAD,), jnp.float32),
        pltpu.VMEM_SHARED((N_NODES, D), jnp.float32),
        pltpu.SemaphoreType.DMA,
    ],
)

_sc_agg = pl.kernel(
    _sc_agg_body,
    out_type=jax.ShapeDtypeStruct((NC, N_NODES, D), jnp.float32),
    mesh=_MESH,
    compiler_params=pltpu.CompilerParams(needs_layout_passes=False),
    scratch_types=[
        pltpu.VMEM((G, CH), jnp.int32),
        pltpu.VMEM((G, CH), jnp.int32),
        pltpu.VMEM((CH, D), jnp.float32),
        pltpu.VMEM_SHARED((N_NODES, D), jnp.float32),
        pltpu.SemaphoreType.DMA,
    ],
)


def _lin_body(relu, agg_ref, cnt_ref, x_ref, wl_ref, wr_ref, b_ref, o_ref):
    aggsum = agg_ref[0] + agg_ref[1]
    cnt = jnp.sum(cnt_ref[...], axis=1)[:, None]
    mean = aggsum / jnp.maximum(cnt, 1.0)
    y = jnp.dot(mean, wl_ref[...], preferred_element_type=jnp.float32)
    y = y + jnp.dot(x_ref[...], wr_ref[...], preferred_element_type=jnp.float32)
    y = y + b_ref[...][None, :]
    if relu:
        y = jnp.maximum(y, 0.0)
    o_ref[...] = y


def _linear(agg, cnt, x, wl, wr, b, relu):
    blk = CBLK
    return pl.pallas_call(
        functools.partial(_lin_body, relu),
        grid=(N_NODES // blk,),
        in_specs=[
            pl.BlockSpec((NC, blk, D), lambda i: (0, i, 0)),
            pl.BlockSpec((blk, NW), lambda i: (i, 0)),
            pl.BlockSpec((blk, D), lambda i: (i, 0)),
            pl.BlockSpec((D, D), lambda i: (0, 0)),
            pl.BlockSpec((D, D), lambda i: (0, 0)),
            pl.BlockSpec((D,), lambda i: (0,)),
        ],
        out_specs=pl.BlockSpec((blk, D), lambda i: (i, 0)),
        out_shape=jax.ShapeDtypeStruct((N_NODES, D), jnp.float32),
    )(agg, cnt, x, wl, wr, b)


def kernel(x, edge_index, W1_l, W1_r, b1, W2_l, W2_r, b2):
    ei = edge_index.astype(jnp.int32)
    srcs = ei[0].reshape(NW, NG, G, CH)
    dsts = ei[1].reshape(NW, NG, G, CH)
    zeros = jnp.zeros((ZCH, D), jnp.float32)
    agg1, cnt = _sc_agg_cnt(x, srcs, dsts, zeros)
    cnt_t = cnt[:, :N_NODES].T
    h = _linear(agg1, cnt_t, x, W1_l, W1_r, b1, relu=True)
    agg2 = _sc_agg(h, srcs, dsts, zeros)
    return _linear(agg2, cnt_t, h, W2_l, W2_r, b2, relu=False)


# speedup vs baseline: 8.8999x; 1.2683x over previous
import functools

import jax
import jax.numpy as jnp
from jax import lax
from jax.experimental import pallas as pl
from jax.experimental.pallas import tpu as pltpu
from jax.experimental.pallas import tpu_sc as plsc

N_NODES = 10000
N_EDGES = 320000
D = 128

NC = 2
NS = 16
NW = NC * NS

CH = 50
G = 10
E_PER_W = N_EDGES // NW
NG = E_PER_W // (G * CH)
ROWS_PER_TILE = N_NODES // NS
ZCH = 125
CBLK = 1000
NPAD = 10240
CSTG = 2000

_MESH = plsc.VectorSubcoreMesh(core_axis_name="c", subcore_axis_name="s")


def _zero_agg(s, zeros_hbm, agg_sh):
    for k in range(ROWS_PER_TILE // ZCH):
        sl = pl.ds(s * ROWS_PER_TILE + k * ZCH, ZCH)
        pltpu.sync_copy(zeros_hbm, agg_sh.at[sl])


def _edge_pipeline(x_hbm, srcs_hbm, dsts_hbm, wid, src_g, dst_g, rows, gsems,
                   ssems, agg_sh):
    def fire_gather(k, p):
        pltpu.async_copy(x_hbm.at[src_g.at[k]], rows[p], gsems[p])

    def wait_gather(p):
        pltpu.make_async_copy(x_hbm.at[src_g.at[0]], rows[p], gsems[p]).wait()

    def fire_scatter(k, p):
        pltpu.async_copy(rows[p], agg_sh.at[dst_g.at[k]], ssems[p], add=True)

    def wait_scatter(p):
        pltpu.make_async_copy(rows[p], agg_sh.at[dst_g.at[0]], ssems[p]).wait()

    def group(g, is_first):
        for k in range(G):
            p = k & 1
            q = 1 - p
            if k < G - 1:
                if not (is_first and k == 0):
                    wait_scatter(q)
                fire_gather(k + 1, q)
                wait_gather(p)
                fire_scatter(k, p)
            else:
                wait_gather(p)
                fire_scatter(k, p)
                if is_first:
                    pltpu.sync_copy(srcs_hbm.at[wid, 1], src_g)
                    pltpu.sync_copy(dsts_hbm.at[wid, 1], dst_g)
                    wait_scatter(q)
                    fire_gather(0, q)
                else:
                    @pl.when(g < NG - 1)
                    def _():
                        pltpu.sync_copy(srcs_hbm.at[wid, g + 1], src_g)
                        pltpu.sync_copy(dsts_hbm.at[wid, g + 1], dst_g)
                        wait_scatter(q)
                        fire_gather(0, q)

    pltpu.sync_copy(srcs_hbm.at[wid, 0], src_g)
    pltpu.sync_copy(dsts_hbm.at[wid, 0], dst_g)
    fire_gather(0, 0)
    group(0, True)

    def gbody(g, carry):
        group(g, False)
        return carry

    lax.fori_loop(1, NG, gbody, 0)
    wait_scatter(0)
    wait_scatter(1)


def _sc_agg_cnt_body(x_hbm, srcs_hbm, dsts_hbm, dflat_hbm, zeros_hbm,
                     out_agg, out_cnt,
                     src_g, dst_g, rows0, rows1, dbuf, cnt_v, agg_sh,
                     gsem0, gsem1, ssem0, ssem1):
    c = lax.axis_index("c")
    s = lax.axis_index("s")
    wid = c * NS + s
    _zero_agg(s, zeros_hbm, agg_sh)
    zeros16 = jnp.zeros((16,), jnp.float32)

    def zstep(v, carry):
        cnt_v[pl.ds(v * 16, 16)] = zeros16
        return carry

    lax.fori_loop(0, NPAD // 16, zstep, 0)
    ones16 = jnp.ones((16,), jnp.float32)
    for t in range(E_PER_W // CSTG):
        pltpu.sync_copy(dflat_hbm.at[wid, t], dbuf)

        def cstep(v, carry):
            idx = dbuf[0, pl.ds(v * 16, 16)]
            plsc.addupdate_scatter(cnt_v, [idx], ones16)
            return carry

        lax.fori_loop(0, CSTG // 16, cstep, 0)
    plsc.subcore_barrier()
    _edge_pipeline(x_hbm, srcs_hbm, dsts_hbm, wid, src_g, dst_g,
                   (rows0, rows1), (gsem0, gsem1), (ssem0, ssem1), agg_sh)
    plsc.subcore_barrier()
    pltpu.sync_copy(cnt_v, out_cnt.at[wid])
    @pl.when(s == 0)
    def _():
        pltpu.sync_copy(agg_sh, out_agg.at[c])


def _sc_agg_body(x_hbm, srcs_hbm, dsts_hbm, zeros_hbm, out_agg,
                 src_g, dst_g, rows0, rows1, agg_sh,
                 gsem0, gsem1, ssem0, ssem1):
    c = lax.axis_index("c")
    s = lax.axis_index("s")
    wid = c * NS + s
    _zero_agg(s, zeros_hbm, agg_sh)
    plsc.subcore_barrier()
    _edge_pipeline(x_hbm, srcs_hbm, dsts_hbm, wid, src_g, dst_g,
                   (rows0, rows1), (gsem0, gsem1), (ssem0, ssem1), agg_sh)
    plsc.subcore_barrier()
    @pl.when(s == 0)
    def _():
        pltpu.sync_copy(agg_sh, out_agg.at[c])


_sc_agg_cnt = pl.kernel(
    _sc_agg_cnt_body,
    out_type=(jax.ShapeDtypeStruct((NC, N_NODES, D), jnp.float32),
              jax.ShapeDtypeStruct((NW, NPAD), jnp.float32)),
    mesh=_MESH,
    compiler_params=pltpu.CompilerParams(needs_layout_passes=False),
    scratch_types=[
        pltpu.VMEM((G, CH), jnp.int32),
        pltpu.VMEM((G, CH), jnp.int32),
        pltpu.VMEM((CH, D), jnp.float32),
        pltpu.VMEM((CH, D), jnp.float32),
        pltpu.VMEM((1, CSTG), jnp.int32),
        pltpu.VMEM((NPAD,), jnp.float32),
        pltpu.VMEM_SHARED((N_NODES, D), jnp.float32),
        pltpu.SemaphoreType.DMA,
        pltpu.SemaphoreType.DMA,
        pltpu.SemaphoreType.DMA,
        pltpu.SemaphoreType.DMA,
    ],
)

_sc_agg = pl.kernel(
    _sc_agg_body,
    out_type=jax.ShapeDtypeStruct((NC, N_NODES, D), jnp.float32),
    mesh=_MESH,
    compiler_params=pltpu.CompilerParams(needs_layout_passes=False),
    scratch_types=[
        pltpu.VMEM((G, CH), jnp.int32),
        pltpu.VMEM((G, CH), jnp.int32),
        pltpu.VMEM((CH, D), jnp.float32),
        pltpu.VMEM((CH, D), jnp.float32),
        pltpu.VMEM_SHARED((N_NODES, D), jnp.float32),
        pltpu.SemaphoreType.DMA,
        pltpu.SemaphoreType.DMA,
        pltpu.SemaphoreType.DMA,
        pltpu.SemaphoreType.DMA,
    ],
)


def _lin_body(relu, agg_ref, cnt_ref, x_ref, wl_ref, wr_ref, b_ref, o_ref):
    aggsum = agg_ref[0] + agg_ref[1]
    cnt = jnp.sum(cnt_ref[...], axis=1)[:, None]
    mean = aggsum / jnp.maximum(cnt, 1.0)
    y = jnp.dot(mean, wl_ref[...], preferred_element_type=jnp.float32)
    y = y + jnp.dot(x_ref[...], wr_ref[...], preferred_element_type=jnp.float32)
    y = y + b_ref[...][None, :]
    if relu:
        y = jnp.maximum(y, 0.0)
    o_ref[...] = y


def _linear(agg, cnt, x, wl, wr, b, relu):
    blk = CBLK
    return pl.pallas_call(
        functools.partial(_lin_body, relu),
        grid=(N_NODES // blk,),
        in_specs=[
            pl.BlockSpec((NC, blk, D), lambda i: (0, i, 0)),
            pl.BlockSpec((blk, NW), lambda i: (i, 0)),
            pl.BlockSpec((blk, D), lambda i: (i, 0)),
            pl.BlockSpec((D, D), lambda i: (0, 0)),
            pl.BlockSpec((D, D), lambda i: (0, 0)),
            pl.BlockSpec((D,), lambda i: (0,)),
        ],
        out_specs=pl.BlockSpec((blk, D), lambda i: (i, 0)),
        out_shape=jax.ShapeDtypeStruct((N_NODES, D), jnp.float32),
    )(agg, cnt, x, wl, wr, b)


def kernel(x, edge_index, W1_l, W1_r, b1, W2_l, W2_r, b2):
    ei = edge_index.astype(jnp.int32)
    srcs = ei[0].reshape(NW, NG, G, CH)
    dsts = ei[1].reshape(NW, NG, G, CH)
    dflat = ei[1].reshape(NW, E_PER_W // CSTG, 1, CSTG)
    zeros = jnp.zeros((ZCH, D), jnp.float32)
    agg1, cnt = _sc_agg_cnt(x, srcs, dsts, dflat, zeros)
    cnt_t = cnt[:, :N_NODES].T
    h = _linear(agg1, cnt_t, x, W1_l, W1_r, b1, relu=True)
    agg2 = _sc_agg(h, srcs, dsts, zeros)
    return _linear(agg2, cnt_t, h, W2_l, W2_r, b2, relu=False)


# speedup vs baseline: 12.3949x; 1.3927x over previous
import functools

import jax
import jax.numpy as jnp
from jax import lax
from jax.experimental import pallas as pl
from jax.experimental.pallas import tpu as pltpu
from jax.experimental.pallas import tpu_sc as plsc

N_NODES = 10000
N_EDGES = 320000
D = 128

NC = 2
NS = 16
NW = NC * NS

CH = 50
G = 25
NBUF = 4
E_PER_W = N_EDGES // NW
NG = E_PER_W // (G * CH)
ROWS_PER_TILE = N_NODES // NS
ZCH = 125
CBLK = 1000
NPAD = 10240

_MESH = plsc.VectorSubcoreMesh(core_axis_name="c", subcore_axis_name="s")


def _zero_agg(s, zeros_hbm, agg_sh):
    for k in range(ROWS_PER_TILE // ZCH):
        sl = pl.ds(s * ROWS_PER_TILE + k * ZCH, ZCH)
        pltpu.sync_copy(zeros_hbm, agg_sh.at[sl])


def _edge_pipeline(x_hbm, srcs_hbm, dsts_hbm, wid, src_g, dst_g, rows, gsems,
                   ssems, agg_sh, count_chunk=None):
    def fire_gather(k, b):
        pltpu.async_copy(x_hbm.at[src_g.at[k]], rows[b], gsems[b])

    def wait_gather(b):
        pltpu.make_async_copy(x_hbm.at[src_g.at[0]], rows[b], gsems[b]).wait()

    def fire_scatter(k, b):
        pltpu.async_copy(rows[b], agg_sh.at[dst_g.at[k]], ssems[b], add=True)

    def wait_scatter(b):
        pltpu.make_async_copy(rows[b], agg_sh.at[dst_g.at[0]], ssems[b]).wait()

    def group(g, is_first):
        for j in range(NBUF - 1):
            if not is_first:
                wait_scatter(j)
            fire_gather(j, j)
        for k in range(G):
            b = k % NBUF
            ahead = k + NBUF - 1
            if ahead < G:
                ab = ahead % NBUF
                if not (is_first and ahead == NBUF - 1):
                    wait_scatter(ab)
                fire_gather(ahead, ab)
            wait_gather(b)
            fire_scatter(k, b)
            if count_chunk is not None:
                count_chunk(k)
        if is_first:
            pltpu.sync_copy(srcs_hbm.at[wid, 1], src_g)
            pltpu.sync_copy(dsts_hbm.at[wid, 1], dst_g)
        else:
            @pl.when(g < NG - 1)
            def _():
                pltpu.sync_copy(srcs_hbm.at[wid, g + 1], src_g)
                pltpu.sync_copy(dsts_hbm.at[wid, g + 1], dst_g)

    pltpu.sync_copy(srcs_hbm.at[wid, 0], src_g)
    pltpu.sync_copy(dsts_hbm.at[wid, 0], dst_g)
    group(0, True)

    def gbody(g, carry):
        group(g, False)
        return carry

    lax.fori_loop(1, NG, gbody, 0)
    for b in range(NBUF):
        wait_scatter(b)


def _sc_agg_cnt_body(x_hbm, srcs_hbm, dsts_hbm, zeros_hbm,
                     out_agg, out_cnt,
                     src_g, dst_g, rows0, rows1, rows2, rows3, cnt_v, agg_sh,
                     gsem0, gsem1, gsem2, gsem3, ssem0, ssem1, ssem2, ssem3):
    c = lax.axis_index("c")
    s = lax.axis_index("s")
    wid = c * NS + s
    _zero_agg(s, zeros_hbm, agg_sh)
    zeros16 = jnp.zeros((16,), jnp.float32)

    def zstep(v, carry):
        cnt_v[pl.ds(v * 16, 16)] = zeros16
        return carry

    lax.fori_loop(0, NPAD // 16, zstep, 0)
    ones16 = jnp.ones((16,), jnp.float32)
    tail_mask = lax.iota(jnp.int32, 16) >= (16 - (CH - 3 * 16))

    def count_chunk(k):
        for off in range(0, 48, 16):
            idx = dst_g[k, pl.ds(off, 16)]
            plsc.addupdate_scatter(cnt_v, [idx], ones16)
        idx = dst_g[k, pl.ds(CH - 16, 16)]
        plsc.addupdate_scatter(cnt_v, [idx], ones16, mask=tail_mask)

    plsc.subcore_barrier()
    _edge_pipeline(x_hbm, srcs_hbm, dsts_hbm, wid, src_g, dst_g,
                   (rows0, rows1, rows2, rows3),
                   (gsem0, gsem1, gsem2, gsem3),
                   (ssem0, ssem1, ssem2, ssem3), agg_sh,
                   count_chunk=count_chunk)
    plsc.subcore_barrier()
    pltpu.sync_copy(cnt_v, out_cnt.at[wid])
    @pl.when(s == 0)
    def _():
        pltpu.sync_copy(agg_sh, out_agg.at[c])


def _sc_agg_body(x_hbm, srcs_hbm, dsts_hbm, zeros_hbm, out_agg,
                 src_g, dst_g, rows0, rows1, rows2, rows3, agg_sh,
                 gsem0, gsem1, gsem2, gsem3, ssem0, ssem1, ssem2, ssem3):
    c = lax.axis_index("c")
    s = lax.axis_index("s")
    wid = c * NS + s
    _zero_agg(s, zeros_hbm, agg_sh)
    plsc.subcore_barrier()
    _edge_pipeline(x_hbm, srcs_hbm, dsts_hbm, wid, src_g, dst_g,
                   (rows0, rows1, rows2, rows3),
                   (gsem0, gsem1, gsem2, gsem3),
                   (ssem0, ssem1, ssem2, ssem3), agg_sh)
    plsc.subcore_barrier()
    @pl.when(s == 0)
    def _():
        pltpu.sync_copy(agg_sh, out_agg.at[c])


_sc_agg_cnt = pl.kernel(
    _sc_agg_cnt_body,
    out_type=(jax.ShapeDtypeStruct((NC, N_NODES, D), jnp.float32),
              jax.ShapeDtypeStruct((NW, NPAD), jnp.float32)),
    mesh=_MESH,
    compiler_params=pltpu.CompilerParams(needs_layout_passes=False),
    scratch_types=(
        [pltpu.VMEM((G, CH), jnp.int32)] * 2
        + [pltpu.VMEM((CH, D), jnp.float32)] * NBUF
        + [pltpu.VMEM((NPAD,), jnp.float32),
           pltpu.VMEM_SHARED((N_NODES, D), jnp.float32)]
        + [pltpu.SemaphoreType.DMA] * (2 * NBUF)
    ),
)

_sc_agg = pl.kernel(
    _sc_agg_body,
    out_type=jax.ShapeDtypeStruct((NC, N_NODES, D), jnp.float32),
    mesh=_MESH,
    compiler_params=pltpu.CompilerParams(needs_layout_passes=False),
    scratch_types=(
        [pltpu.VMEM((G, CH), jnp.int32)] * 2
        + [pltpu.VMEM((CH, D), jnp.float32)] * NBUF
        + [pltpu.VMEM_SHARED((N_NODES, D), jnp.float32)]
        + [pltpu.SemaphoreType.DMA] * (2 * NBUF)
    ),
)


def _lin_body(relu, agg_ref, cnt_ref, x_ref, wl_ref, wr_ref, b_ref, o_ref):
    aggsum = agg_ref[0] + agg_ref[1]
    cnt = jnp.sum(cnt_ref[...], axis=1)[:, None]
    mean = aggsum / jnp.maximum(cnt, 1.0)
    y = jnp.dot(mean, wl_ref[...], preferred_element_type=jnp.float32)
    y = y + jnp.dot(x_ref[...], wr_ref[...], preferred_element_type=jnp.float32)
    y = y + b_ref[...][None, :]
    if relu:
        y = jnp.maximum(y, 0.0)
    o_ref[...] = y


def _linear(agg, cnt, x, wl, wr, b, relu):
    blk = CBLK
    return pl.pallas_call(
        functools.partial(_lin_body, relu),
        grid=(N_NODES // blk,),
        in_specs=[
            pl.BlockSpec((NC, blk, D), lambda i: (0, i, 0)),
            pl.BlockSpec((blk, NW), lambda i: (i, 0)),
            pl.BlockSpec((blk, D), lambda i: (i, 0)),
            pl.BlockSpec((D, D), lambda i: (0, 0)),
            pl.BlockSpec((D, D), lambda i: (0, 0)),
            pl.BlockSpec((D,), lambda i: (0,)),
        ],
        out_specs=pl.BlockSpec((blk, D), lambda i: (i, 0)),
        out_shape=jax.ShapeDtypeStruct((N_NODES, D), jnp.float32),
    )(agg, cnt, x, wl, wr, b)


def kernel(x, edge_index, W1_l, W1_r, b1, W2_l, W2_r, b2):
    ei = edge_index.astype(jnp.int32)
    srcs = ei[0].reshape(NW, NG, G, CH)
    dsts = ei[1].reshape(NW, NG, G, CH)
    zeros = jnp.zeros((ZCH, D), jnp.float32)
    agg1, cnt = _sc_agg_cnt(x, srcs, dsts, zeros)
    cnt_t = cnt[:, :N_NODES].T
    h = _linear(agg1, cnt_t, x, W1_l, W1_r, b1, relu=True)
    agg2 = _sc_agg(h, srcs, dsts, zeros)
    return _linear(agg2, cnt_t, h, W2_l, W2_r, b2, relu=False)


# speedup vs baseline: 12.4856x; 1.0073x over previous
import functools

import jax
import jax.numpy as jnp
from jax import lax
from jax.experimental import pallas as pl
from jax.experimental.pallas import tpu as pltpu
from jax.experimental.pallas import tpu_sc as plsc

N_NODES = 10000
N_EDGES = 320000
D = 128

NC = 2
NS = 16
NW = NC * NS

CH = 50
G = 25
NBUF1 = 4
NBUF2 = 5
E_PER_W = N_EDGES // NW
NG = E_PER_W // (G * CH)
ROWS_PER_TILE = N_NODES // NS
ZCH = 125
CBLK = 1000
NPAD = 10240

_MESH = plsc.VectorSubcoreMesh(core_axis_name="c", subcore_axis_name="s")


def _zero_agg(s, zeros_hbm, agg_sh):
    for k in range(ROWS_PER_TILE // ZCH):
        sl = pl.ds(s * ROWS_PER_TILE + k * ZCH, ZCH)
        pltpu.sync_copy(zeros_hbm, agg_sh.at[sl])


def _edge_pipeline(x_hbm, srcs_hbm, dsts_hbm, wid, src_g, dst_g, rows, gsems,
                   ssems, agg_sh, count_chunk=None):
    NBUF = len(rows)

    def fire_gather(k, b):
        pltpu.async_copy(x_hbm.at[src_g.at[k]], rows[b], gsems[b])

    def wait_gather(b):
        pltpu.make_async_copy(x_hbm.at[src_g.at[0]], rows[b], gsems[b]).wait()

    def fire_scatter(k, b):
        pltpu.async_copy(rows[b], agg_sh.at[dst_g.at[k]], ssems[b], add=True)

    def wait_scatter(b):
        pltpu.make_async_copy(rows[b], agg_sh.at[dst_g.at[0]], ssems[b]).wait()

    def group(g, is_first):
        for j in range(NBUF - 1):
            if not is_first:
                wait_scatter(j)
            fire_gather(j, j)
        for k in range(G):
            b = k % NBUF
            ahead = k + NBUF - 1
            if ahead < G:
                ab = ahead % NBUF
                if not (is_first and ahead == NBUF - 1):
                    wait_scatter(ab)
                fire_gather(ahead, ab)
            wait_gather(b)
            fire_scatter(k, b)
            if count_chunk is not None:
                count_chunk(k)
        if is_first:
            pltpu.sync_copy(srcs_hbm.at[wid, 1], src_g)
            pltpu.sync_copy(dsts_hbm.at[wid, 1], dst_g)
        else:
            @pl.when(g < NG - 1)
            def _():
                pltpu.sync_copy(srcs_hbm.at[wid, g + 1], src_g)
                pltpu.sync_copy(dsts_hbm.at[wid, g + 1], dst_g)

    pltpu.sync_copy(srcs_hbm.at[wid, 0], src_g)
    pltpu.sync_copy(dsts_hbm.at[wid, 0], dst_g)
    group(0, True)

    def gbody(g, carry):
        group(g, False)
        return carry

    lax.fori_loop(1, NG, gbody, 0)
    for b in range(NBUF):
        wait_scatter(b)


def _sc_agg_cnt_body(x_hbm, srcs_hbm, dsts_hbm, zeros_hbm,
                     out_agg, out_cnt, src_g, dst_g, *bufs):
    rows = bufs[:NBUF1]
    cnt_v, agg_sh = bufs[NBUF1], bufs[NBUF1 + 1]
    gsems = bufs[NBUF1 + 2:2 * NBUF1 + 2]
    ssems = bufs[2 * NBUF1 + 2:]
    c = lax.axis_index("c")
    s = lax.axis_index("s")
    wid = c * NS + s
    _zero_agg(s, zeros_hbm, agg_sh)
    zeros16 = jnp.zeros((16,), jnp.float32)

    def zstep(v, carry):
        cnt_v[pl.ds(v * 16, 16)] = zeros16
        return carry

    lax.fori_loop(0, NPAD // 16, zstep, 0)
    ones16 = jnp.ones((16,), jnp.float32)
    tail_mask = lax.iota(jnp.int32, 16) >= (16 - (CH - 3 * 16))

    def count_chunk(k):
        for off in range(0, 48, 16):
            idx = dst_g[k, pl.ds(off, 16)]
            plsc.addupdate_scatter(cnt_v, [idx], ones16)
        idx = dst_g[k, pl.ds(CH - 16, 16)]
        plsc.addupdate_scatter(cnt_v, [idx], ones16, mask=tail_mask)

    plsc.subcore_barrier()
    _edge_pipeline(x_hbm, srcs_hbm, dsts_hbm, wid, src_g, dst_g,
                   rows, gsems, ssems, agg_sh, count_chunk=count_chunk)
    plsc.subcore_barrier()
    pltpu.sync_copy(cnt_v, out_cnt.at[wid])
    @pl.when(s == 0)
    def _():
        pltpu.sync_copy(agg_sh, out_agg.at[c])


def _sc_agg_body(x_hbm, srcs_hbm, dsts_hbm, zeros_hbm, out_agg,
                 src_g, dst_g, *bufs):
    rows = bufs[:NBUF2]
    agg_sh = bufs[NBUF2]
    gsems = bufs[NBUF2 + 1:2 * NBUF2 + 1]
    ssems = bufs[2 * NBUF2 + 1:]
    c = lax.axis_index("c")
    s = lax.axis_index("s")
    wid = c * NS + s
    _zero_agg(s, zeros_hbm, agg_sh)
    plsc.subcore_barrier()
    _edge_pipeline(x_hbm, srcs_hbm, dsts_hbm, wid, src_g, dst_g,
                   rows, gsems, ssems, agg_sh)
    plsc.subcore_barrier()
    @pl.when(s == 0)
    def _():
        pltpu.sync_copy(agg_sh, out_agg.at[c])


_sc_agg_cnt = pl.kernel(
    _sc_agg_cnt_body,
    out_type=(jax.ShapeDtypeStruct((NC, N_NODES, D), jnp.float32),
              jax.ShapeDtypeStruct((NW, NPAD), jnp.float32)),
    mesh=_MESH,
    compiler_params=pltpu.CompilerParams(needs_layout_passes=False),
    scratch_types=(
        [pltpu.VMEM((G, CH), jnp.int32)] * 2
        + [pltpu.VMEM((CH, D), jnp.float32)] * NBUF1
        + [pltpu.VMEM((NPAD,), jnp.float32),
           pltpu.VMEM_SHARED((N_NODES, D), jnp.float32)]
        + [pltpu.SemaphoreType.DMA] * (2 * NBUF1)
    ),
)

_sc_agg = pl.kernel(
    _sc_agg_body,
    out_type=jax.ShapeDtypeStruct((NC, N_NODES, D), jnp.float32),
    mesh=_MESH,
    compiler_params=pltpu.CompilerParams(needs_layout_passes=False),
    scratch_types=(
        [pltpu.VMEM((G, CH), jnp.int32)] * 2
        + [pltpu.VMEM((CH, D), jnp.float32)] * NBUF2
        + [pltpu.VMEM_SHARED((N_NODES, D), jnp.float32)]
        + [pltpu.SemaphoreType.DMA] * (2 * NBUF2)
    ),
)


def _lin_body(relu, agg_ref, cnt_ref, x_ref, wl_ref, wr_ref, b_ref, o_ref):
    aggsum = agg_ref[0] + agg_ref[1]
    cnt = jnp.sum(cnt_ref[...], axis=1)[:, None]
    mean = aggsum / jnp.maximum(cnt, 1.0)
    y = jnp.dot(mean, wl_ref[...], preferred_element_type=jnp.float32)
    y = y + jnp.dot(x_ref[...], wr_ref[...], preferred_element_type=jnp.float32)
    y = y + b_ref[...][None, :]
    if relu:
        y = jnp.maximum(y, 0.0)
    o_ref[...] = y


def _linear(agg, cnt, x, wl, wr, b, relu):
    blk = CBLK
    return pl.pallas_call(
        functools.partial(_lin_body, relu),
        grid=(N_NODES // blk,),
        in_specs=[
            pl.BlockSpec((NC, blk, D), lambda i: (0, i, 0)),
            pl.BlockSpec((blk, NW), lambda i: (i, 0)),
            pl.BlockSpec((blk, D), lambda i: (i, 0)),
            pl.BlockSpec((D, D), lambda i: (0, 0)),
            pl.BlockSpec((D, D), lambda i: (0, 0)),
            pl.BlockSpec((D,), lambda i: (0,)),
        ],
        out_specs=pl.BlockSpec((blk, D), lambda i: (i, 0)),
        out_shape=jax.ShapeDtypeStruct((N_NODES, D), jnp.float32),
    )(agg, cnt, x, wl, wr, b)


def kernel(x, edge_index, W1_l, W1_r, b1, W2_l, W2_r, b2):
    ei = edge_index.astype(jnp.int32)
    srcs = ei[0].reshape(NW, NG, G, CH)
    dsts = ei[1].reshape(NW, NG, G, CH)
    zeros = jnp.zeros((ZCH, D), jnp.float32)
    agg1, cnt = _sc_agg_cnt(x, srcs, dsts, zeros)
    cnt_t = cnt[:, :N_NODES].T
    h = _linear(agg1, cnt_t, x, W1_l, W1_r, b1, relu=True)
    agg2 = _sc_agg(h, srcs, dsts, zeros)
    return _linear(agg2, cnt_t, h, W2_l, W2_r, b2, relu=False)
